# baseline (speedup 1.0000x reference)
import functools

import jax
import jax.numpy as jnp
import numpy as np
from jax import lax
from jax.experimental import pallas as pl
from jax.experimental.pallas import tpu as pltpu
from jax.experimental.pallas import tpu_sc as plsc

_N_LEVELS = 65536
_LANES = 16
_BASIS_ROW = np.ascontiguousarray(
    ((2.0 ** np.arange(16, dtype=np.float64)) / (2.0 ** 16)).astype(np.float32)[None, :]
)
_I8 = np.eye(8, dtype=np.float32)
_I4 = np.eye(4, dtype=np.float32)


def _table_body(x_ref, w1_ref, b1_ref, w2_ref, b2_ref, w3_ref, b3_ref, w4_ref,
                b4_ref, i8_ref, i4_ref, basis_ref, out_ref, idx_ref):
    xv = x_ref[...]
    t = (xv * 65536.0 + 65535.0) * 0.5 + 0.5
    k0 = jnp.clip(t.astype(jnp.int32), 0, _N_LEVELS - 1)
    km = jnp.maximum(k0 - 1, 0)
    kp = jnp.minimum(k0 + 1, _N_LEVELS - 1)
    dm = jnp.abs(xv - (km * 2 - 65535).astype(jnp.float32) * (1.0 / 65536.0))
    dc = jnp.abs(xv - (k0 * 2 - 65535).astype(jnp.float32) * (1.0 / 65536.0))
    dp = jnp.abs(xv - (kp * 2 - 65535).astype(jnp.float32) * (1.0 / 65536.0))
    idx_ref[...] = jnp.where((dm <= dc) & (dm <= dp), km,
                             jnp.where(dc <= dp, k0, kp))
    jshift = lax.broadcasted_iota(jnp.int32, (16, 1), 0)
    i = lax.broadcasted_iota(jnp.int32, (16, _N_LEVELS), 1)
    signbit = (~i << (31 - jshift)) & jnp.int32(-2147483648)
    bits_t = lax.bitcast_convert_type(signbit | jnp.int32(0x3F800000),
                                      jnp.float32)
    b1c = jax.lax.dot_general(i8_ref[...], b1_ref[...].reshape(1, 8),
                              (((1,), (1,)), ((), ())),
                              preferred_element_type=jnp.float32)
    b2c = jax.lax.dot_general(i4_ref[...], b2_ref[...].reshape(1, 4),
                              (((1,), (1,)), ((), ())),
                              preferred_element_type=jnp.float32)
    h = jnp.maximum(
        jnp.dot(w1_ref[...], bits_t, preferred_element_type=jnp.float32)
        + b1c, 0.0)
    z = jnp.dot(w2_ref[...], h, preferred_element_type=jnp.float32) + b2c
    s = jnp.sign(z)
    basis = basis_ref[...]
    bh = jnp.maximum(
        jax.lax.dot_general(basis, w3_ref[...], (((1,), (1,)), ((), ())),
                            preferred_element_type=jnp.float32)
        + b3_ref[...].reshape(1, 8), 0.0)
    xb = 1.0 / (1.0 + jnp.exp(
        -(jax.lax.dot_general(bh, w4_ref[...], (((1,), (1,)), ((), ())),
                              preferred_element_type=jnp.float32)
          + b4_ref[...].reshape(1, 4))))
    tab = jnp.dot(xb, s, preferred_element_type=jnp.float32)
    out_ref[...] = tab.reshape(_N_LEVELS)


def _table_and_idx(x, W1, b1, W2, b2, W3, b3, W4, b4):
    return pl.pallas_call(
        _table_body,
        out_shape=(jax.ShapeDtypeStruct((_N_LEVELS,), jnp.float32),
                   jax.ShapeDtypeStruct(x.shape, jnp.int32)),
    )(x, W1, b1, W2, b2, W3, b3, W4, b4,
      jnp.asarray(_I8), jnp.asarray(_I4), jnp.asarray(_BASIS_ROW))


def _sc_lookup(idx, table):
    info = plsc.get_sparse_core_info()
    num_cores = info.num_cores
    nw = info.num_cores * info.num_subcores
    nrow, ncol = idx.shape
    assert nrow % nw == 0 and ncol % 128 == 0
    rpw = nrow // nw
    mesh = plsc.VectorSubcoreMesh(core_axis_name="c", subcore_axis_name="s")

    @functools.partial(
        pl.kernel,
        mesh=mesh,
        compiler_params=pltpu.CompilerParams(needs_layout_passes=False),
        out_type=jax.ShapeDtypeStruct((nrow, ncol), jnp.float32),
        scratch_types=[
            pltpu.VMEM((rpw, ncol), jnp.int32),
            pltpu.VMEM((rpw, ncol), jnp.float32),
            pltpu.SemaphoreType.DMA((rpw,)),
            pltpu.SemaphoreType.DMA((rpw,)),
            pltpu.SemaphoreType.DMA,
        ],
    )
    def _body(idx_hbm, tab_hbm, out_hbm, idx_v, out_v, isem, gsem, osem):
        wid = lax.axis_index("s") * num_cores + lax.axis_index("c")
        base = wid * rpw
        idx_cps = [
            pltpu.async_copy(idx_hbm.at[pl.ds(base + r, 1)],
                             idx_v.at[pl.ds(r, 1)], isem.at[r])
            for r in range(rpw)
        ]
        gather_cps = []
        for r in range(rpw):
            idx_cps[r].wait()
            gather_cps.append([
                pltpu.async_copy(tab_hbm.at[idx_v.at[r, pl.ds(c, 128)]],
                                 out_v.at[r, pl.ds(c, 128)], gsem.at[r])
                for c in range(0, ncol, 128)
            ])
        out_cps = []
        for r in range(rpw):
            for cp in gather_cps[r]:
                cp.wait()
            out_cps.append(
                pltpu.async_copy(out_v.at[pl.ds(r, 1)],
                                 out_hbm.at[pl.ds(base + r, 1)], osem))
        for cp in out_cps:
            cp.wait()

    return _body(idx, table)


def kernel(x, W1, b1, W2, b2, W3, b3, W4, b4):
    table, idx = _table_and_idx(x, W1, b1, W2, b2, W3, b3, W4, b4)
    return _sc_lookup(idx, table)


# speedup vs baseline: 1.0102x; 1.0102x over previous
import functools

import jax
import jax.numpy as jnp
import numpy as np
from jax import lax
from jax.experimental import pallas as pl
from jax.experimental.pallas import tpu as pltpu
from jax.experimental.pallas import tpu_sc as plsc

_N_LEVELS = 65536
_LANES = 16
_BASIS_ROW = np.ascontiguousarray(
    ((2.0 ** np.arange(16, dtype=np.float64)) / (2.0 ** 16)).astype(np.float32)[None, :]
)
_I8 = np.eye(8, dtype=np.float32)
_I4 = np.eye(4, dtype=np.float32)


def _table_body(x_ref, w1_ref, b1_ref, w2_ref, b2_ref, w3_ref, b3_ref, w4_ref,
                b4_ref, i8_ref, i4_ref, basis_ref, out_ref, idx_ref):
    xv = x_ref[...]
    t = (xv * 65536.0 + 65535.0) * 0.5 + 0.5
    k0 = jnp.clip(t.astype(jnp.int32), 0, _N_LEVELS - 1)
    km = jnp.maximum(k0 - 1, 0)
    kp = jnp.minimum(k0 + 1, _N_LEVELS - 1)
    dm = jnp.abs(xv - (km * 2 - 65535).astype(jnp.float32) * (1.0 / 65536.0))
    dc = jnp.abs(xv - (k0 * 2 - 65535).astype(jnp.float32) * (1.0 / 65536.0))
    dp = jnp.abs(xv - (kp * 2 - 65535).astype(jnp.float32) * (1.0 / 65536.0))
    idx_ref[...] = jnp.where((dm <= dc) & (dm <= dp), km,
                             jnp.where(dc <= dp, k0, kp))
    jshift = lax.broadcasted_iota(jnp.int32, (16, 1), 0)
    i = lax.broadcasted_iota(jnp.int32, (16, _N_LEVELS), 1)
    signbit = (~i << (31 - jshift)) & jnp.int32(-2147483648)
    bits_t = lax.bitcast_convert_type(signbit | jnp.int32(0x3F800000),
                                      jnp.float32)
    b1c = jax.lax.dot_general(i8_ref[...], b1_ref[...].reshape(1, 8),
                              (((1,), (1,)), ((), ())),
                              preferred_element_type=jnp.float32)
    b2c = jax.lax.dot_general(i4_ref[...], b2_ref[...].reshape(1, 4),
                              (((1,), (1,)), ((), ())),
                              preferred_element_type=jnp.float32)
    h = jnp.maximum(
        jnp.dot(w1_ref[...], bits_t, preferred_element_type=jnp.float32)
        + b1c, 0.0)
    z = jnp.dot(w2_ref[...], h, preferred_element_type=jnp.float32) + b2c
    s = jnp.sign(z)
    basis = basis_ref[...]
    bh = jnp.maximum(
        jax.lax.dot_general(basis, w3_ref[...], (((1,), (1,)), ((), ())),
                            preferred_element_type=jnp.float32)
        + b3_ref[...].reshape(1, 8), 0.0)
    xb = 1.0 / (1.0 + jnp.exp(
        -(jax.lax.dot_general(bh, w4_ref[...], (((1,), (1,)), ((), ())),
                              preferred_element_type=jnp.float32)
          + b4_ref[...].reshape(1, 4))))
    tab = jnp.dot(xb, s, preferred_element_type=jnp.float32)
    out_ref[...] = tab.reshape(_N_LEVELS)


def _table_and_idx(x, W1, b1, W2, b2, W3, b3, W4, b4):
    return pl.pallas_call(
        _table_body,
        out_shape=(jax.ShapeDtypeStruct((_N_LEVELS,), jnp.float32),
                   jax.ShapeDtypeStruct(x.shape, jnp.int32)),
    )(x, W1, b1, W2, b2, W3, b3, W4, b4,
      jnp.asarray(_I8), jnp.asarray(_I4), jnp.asarray(_BASIS_ROW))


def _sc_lookup(idx, table):
    info = plsc.get_sparse_core_info()
    num_cores = info.num_cores
    nw = info.num_cores * info.num_subcores
    nrow, ncol = idx.shape
    assert nrow % nw == 0 and ncol % 128 == 0
    rpw = nrow // nw
    mesh = plsc.VectorSubcoreMesh(core_axis_name="c", subcore_axis_name="s")

    @functools.partial(
        pl.kernel,
        mesh=mesh,
        compiler_params=pltpu.CompilerParams(needs_layout_passes=False),
        out_type=jax.ShapeDtypeStruct((nrow, ncol), jnp.float32),
        scratch_types=[
            pltpu.VMEM((rpw, ncol), jnp.int32),
            pltpu.VMEM((rpw, ncol), jnp.float32),
            pltpu.SemaphoreType.DMA,
        ],
    )
    def _body(idx_hbm, tab_hbm, out_hbm, idx_v, out_v, sem):
        wid = lax.axis_index("s") * num_cores + lax.axis_index("c")
        base = wid * rpw
        pltpu.sync_copy(idx_hbm.at[pl.ds(base, rpw)], idx_v)
        copies = [
            pltpu.async_copy(tab_hbm.at[idx_v.at[r, pl.ds(c, 128)]],
                             out_v.at[r, pl.ds(c, 128)], sem)
            for r in range(rpw)
            for c in range(0, ncol, 128)
        ]
        for cp in copies:
            cp.wait()
        pltpu.sync_copy(out_v, out_hbm.at[pl.ds(base, rpw)])

    return _body(idx, table)


def kernel(x, W1, b1, W2, b2, W3, b3, W4, b4):
    table, idx = _table_and_idx(x, W1, b1, W2, b2, W3, b3, W4, b4)
    return _sc_lookup(idx, table)
